# Initial kernel scaffold; baseline (speedup 1.0000x reference)
import jax
import jax.numpy as jnp
from jax.experimental import pallas as pl


def kernel(x_paper, x_author, edge_index_writes, edge_index_cites, W_kqv_paper, b_kqv_paper, W_kqv_author, b_kqv_author, W_k_rel, W_v_rel, W_out_paper, b_out_paper, skip_paper, p_rel_writes, p_rel_cites, W_cls, b_cls):
    raise NotImplementedError("write your pallas kernel here")


# speedup vs baseline: 4.1455x; 4.1455x over previous
import dataclasses
import functools
import math

import jax
import jax.numpy as jnp
from jax import lax
from jax.experimental import pallas as pl
from jax.experimental.pallas import tpu as pltpu
from jax.experimental.pallas import tpu_sc as plsc

N_PAPER = 6000
N_AUTHOR = 4000
N_TOT = N_PAPER + N_AUTHOR
F = 128
H = 4
DH = 32
E = 320000
NUM_CLASSES = 3

NC = 2
NS = 16
NW = NC * NS
CHUNK = 80
KBUF = 5
CHUNKS_PER_W = E // (CHUNK * NW)
ACC_ROWS = 6016
ROWS_PER_TILE = ACC_ROWS // NS
SCHUNK = 40
SKBUF = 2
SCHUNKS_PER_W = E // (SCHUNK * NW)

_sc_mesh = plsc.VectorSubcoreMesh(core_axis_name="c", subcore_axis_name="s")

_sc_params = pltpu.CompilerParams()
if "needs_layout_passes" in pltpu.CompilerParams.__dataclass_fields__:
    _sc_params = dataclasses.replace(_sc_params, needs_layout_passes=False)


def _proj_body(xp_ref, xa_ref, wp_ref, bp_ref, wa_ref, ba_ref, wkr_ref,
               wvr_ref, prw_ref, prc_ref, qp_ref, k_ref, v_ref):
    xp = xp_ref[...]
    xa = xa_ref[...]
    wp = wp_ref[...]
    bp = bp_ref[...]
    wa = wa_ref[...]
    ba = ba_ref[...]
    wkr = wkr_ref[...]
    wvr = wvr_ref[...]
    prw = prw_ref[...]
    prc = prc_ref[...]

    scale = 1.0 / math.sqrt(DH)
    qp_ref[...] = (jnp.dot(xp, wp[:, F:2 * F],
                           preferred_element_type=jnp.float32)
                   + bp[:, F:2 * F]) * scale

    def fold(w, b, rel, et, pr):
        cols, bcols = [], []
        for h in range(H):
            r = rel[2 * h + et]
            wh = jnp.dot(w[:, h * DH:(h + 1) * DH], r,
                         preferred_element_type=jnp.float32)
            bh = jnp.dot(b[:, h * DH:(h + 1) * DH], r,
                         preferred_element_type=jnp.float32)
            if pr is not None:
                s = pr[0:1, h:h + 1]
                wh = wh * s
                bh = bh * s
            cols.append(wh)
            bcols.append(bh)
        return jnp.concatenate(cols, axis=1), jnp.concatenate(bcols, axis=1)

    wka, bka = fold(wa[:, :F], ba[:, :F], wkr, 0, prw)
    wkp, bkp = fold(wp[:, :F], bp[:, :F], wkr, 1, prc)
    wva, bva = fold(wa[:, 2 * F:], ba[:, 2 * F:], wvr, 0, None)
    wvp, bvp = fold(wp[:, 2 * F:], bp[:, 2 * F:], wvr, 1, None)

    k_ref[:N_AUTHOR, :] = jnp.dot(xa, wka,
                                  preferred_element_type=jnp.float32) + bka
    k_ref[N_AUTHOR:, :] = jnp.dot(xp, wkp,
                                  preferred_element_type=jnp.float32) + bkp
    v_ref[:N_AUTHOR, :] = jnp.dot(xa, wva,
                                  preferred_element_type=jnp.float32) + bva
    v_ref[N_AUTHOR:, :] = jnp.dot(xp, wvp,
                                  preferred_element_type=jnp.float32) + bvp


_proj = pl.pallas_call(
    _proj_body,
    out_shape=[
        jax.ShapeDtypeStruct((N_PAPER, F), jnp.float32),
        jax.ShapeDtypeStruct((N_TOT, F), jnp.float32),
        jax.ShapeDtypeStruct((N_TOT, F), jnp.float32),
    ],
)


GW = 80


def _sc_gather(table, idx2d):
    @functools.partial(pl.kernel,
                       out_type=jax.ShapeDtypeStruct((E, F), jnp.float32),
                       mesh=_sc_mesh)
    def body(table_hbm, idx_hbm, out_hbm):
        def step(i_vmem, o_vmem):
            pltpu.sync_copy(table_hbm.at[i_vmem.at[0]], o_vmem)

        pltpu.emit_pipeline(
            step,
            grid=(E // GW,),
            in_specs=[pl.BlockSpec((1, GW), lambda i: (i, 0))],
            out_specs=[pl.BlockSpec((GW, F), lambda i: (i, 0))],
            core_axis_name=("c", "s"),
            dimension_semantics=(pltpu.PARALLEL,),
        )(idx_hbm, out_hbm)

    return body(table, idx2d)


_BC = 4000


def _edge_body(qi_ref, kj_ref, vj_ref, m_ref, mt_ref, msg_ref, exb_ref):
    p = qi_ref[...] * kj_ref[...]
    ex = jnp.exp(jnp.dot(p, m_ref[...], preferred_element_type=jnp.float32))
    exb = jnp.dot(ex, mt_ref[...], preferred_element_type=jnp.float32)
    msg_ref[...] = exb * vj_ref[...]
    exb_ref[...] = exb


_edge = pl.pallas_call(
    _edge_body,
    grid=(E // _BC,),
    in_specs=[
        pl.BlockSpec((_BC, F), lambda i: (i, 0)),
        pl.BlockSpec((_BC, F), lambda i: (i, 0)),
        pl.BlockSpec((_BC, F), lambda i: (i, 0)),
        pl.BlockSpec((F, H), lambda i: (0, 0)),
        pl.BlockSpec((H, F), lambda i: (0, 0)),
    ],
    out_specs=[
        pl.BlockSpec((_BC, F), lambda i: (i, 0)),
        pl.BlockSpec((_BC, F), lambda i: (i, 0)),
    ],
    out_shape=[
        jax.ShapeDtypeStruct((E, F), jnp.float32),
        jax.ShapeDtypeStruct((E, F), jnp.float32),
    ],
)


SW = 40


def _sc_scatter(msg, exb, dst2d, zrows):
    scratch = [pltpu.VMEM_SHARED((ACC_ROWS, F), jnp.float32),
               pltpu.VMEM_SHARED((ACC_ROWS, F), jnp.float32)]

    @functools.partial(
        pl.kernel,
        out_type=[
            jax.ShapeDtypeStruct((NC, ACC_ROWS, F), jnp.float32),
            jax.ShapeDtypeStruct((NC, ACC_ROWS, F), jnp.float32),
        ],
        mesh=_sc_mesh,
        scratch_types=scratch)
    def body(msg_hbm, exb_hbm, idx_hbm, z_hbm, out_hbm, outs_hbm, acc,
             acc_s):
        sid = lax.axis_index("s")
        cid = lax.axis_index("c")

        pltpu.sync_copy(z_hbm,
                        acc.at[pl.ds(sid * ROWS_PER_TILE, ROWS_PER_TILE)])
        pltpu.sync_copy(z_hbm,
                        acc_s.at[pl.ds(sid * ROWS_PER_TILE, ROWS_PER_TILE)])
        plsc.subcore_barrier()

        def step(i_vmem, m_vmem, e_vmem):
            pltpu.sync_copy(m_vmem, acc.at[i_vmem.at[0]], add=True)
            pltpu.sync_copy(e_vmem, acc_s.at[i_vmem.at[0]], add=True)

        pltpu.emit_pipeline(
            step,
            grid=(E // SW,),
            in_specs=[pl.BlockSpec((1, SW), lambda i: (i, 0)),
                      pl.BlockSpec((SW, F), lambda i: (i, 0)),
                      pl.BlockSpec((SW, F), lambda i: (i, 0))],
            out_specs=[],
            core_axis_name=("c", "s"),
            dimension_semantics=(pltpu.PARALLEL,),
        )(idx_hbm, msg_hbm, exb_hbm)

        plsc.subcore_barrier()
        pltpu.sync_copy(acc.at[pl.ds(sid * ROWS_PER_TILE, ROWS_PER_TILE)],
                        out_hbm.at[cid, pl.ds(sid * ROWS_PER_TILE,
                                              ROWS_PER_TILE)])
        pltpu.sync_copy(acc_s.at[pl.ds(sid * ROWS_PER_TILE, ROWS_PER_TILE)],
                        outs_hbm.at[cid, pl.ds(sid * ROWS_PER_TILE,
                                               ROWS_PER_TILE)])

    return body(msg, exb, dst2d, zrows)


def _final_body(acc_ref, s_ref, xp_ref, wout_ref, bout_ref, skip_ref,
                wcls_ref, bcls_ref, out_ref):
    num = acc_ref[0, :N_PAPER, :] + acc_ref[1, :N_PAPER, :]
    den = s_ref[0, :N_PAPER, :] + s_ref[1, :N_PAPER, :]
    agg = num / (den + 1e-16)
    gel = 0.5 * agg * (1.0 + lax.erf(agg * (1.0 / math.sqrt(2.0))))
    hh = jnp.dot(gel, wout_ref[...],
                 preferred_element_type=jnp.float32) + bout_ref[...]
    al = jax.nn.sigmoid(skip_ref[...])
    o2 = al * hh + (1.0 - al) * xp_ref[...]
    out_ref[...] = jnp.dot(o2, wcls_ref[...],
                           preferred_element_type=jnp.float32) + bcls_ref[...]


_final = pl.pallas_call(
    _final_body,
    out_shape=jax.ShapeDtypeStruct((N_PAPER, NUM_CLASSES), jnp.float32),
)


def kernel(x_paper, x_author, edge_index_writes, edge_index_cites,
           W_kqv_paper, b_kqv_paper, W_kqv_author, b_kqv_author, W_k_rel,
           W_v_rel, W_out_paper, b_out_paper, skip_paper, p_rel_writes,
           p_rel_cites, W_cls, b_cls):
    src = jnp.concatenate([edge_index_writes[0],
                           edge_index_cites[0] + N_AUTHOR])
    dst = jnp.concatenate([edge_index_writes[1], edge_index_cites[1]])

    qp, kcat, vcat = _proj(x_paper, x_author, W_kqv_paper,
                           b_kqv_paper.reshape(1, -1), W_kqv_author,
                           b_kqv_author.reshape(1, -1), W_k_rel, W_v_rel,
                           p_rel_writes, p_rel_cites)

    dst_g = dst.reshape(E // GW, GW)
    src_g = src.reshape(E // GW, GW)
    qi = _sc_gather(qp, dst_g)
    kj = _sc_gather(kcat, src_g)
    vj = _sc_gather(vcat, src_g)

    m = (jnp.arange(F)[:, None] // DH ==
         jnp.arange(H)[None, :]).astype(jnp.float32)
    msg, exb = _edge(qi, kj, vj, m, m.T)

    a0 = jax.ops.segment_sum(msg, dst, num_segments=ACC_ROWS)
    s0 = jax.ops.segment_sum(exb, dst, num_segments=ACC_ROWS)
    acc = jnp.stack([a0, jnp.zeros_like(a0)])
    acc_s = jnp.stack([s0, jnp.zeros_like(s0)])

    return _final(acc, acc_s, x_paper, W_out_paper,
                  b_out_paper.reshape(1, -1),
                  jnp.broadcast_to(skip_paper.reshape(1, 1), (1, F)),
                  W_cls, b_cls.reshape(1, -1))


# speedup vs baseline: 9.6061x; 2.3172x over previous
import dataclasses
import functools
import math

import jax
import jax.numpy as jnp
from jax import lax
from jax.experimental import pallas as pl
from jax.experimental.pallas import tpu as pltpu
from jax.experimental.pallas import tpu_sc as plsc

N_PAPER = 6000
N_AUTHOR = 4000
N_TOT = N_PAPER + N_AUTHOR
F = 128
H = 4
DH = 32
E = 320000
NUM_CLASSES = 3

NC = 2
NS = 16
NW = NC * NS
CHUNK = 80
KBUF = 5
CHUNKS_PER_W = E // (CHUNK * NW)
ACC_ROWS = 6016
ROWS_PER_TILE = ACC_ROWS // NS
SCHUNK = 40
SKBUF = 2
SCHUNKS_PER_W = E // (SCHUNK * NW)

_sc_mesh = plsc.VectorSubcoreMesh(core_axis_name="c", subcore_axis_name="s")

_sc_params = pltpu.CompilerParams()
if "needs_layout_passes" in pltpu.CompilerParams.__dataclass_fields__:
    _sc_params = dataclasses.replace(_sc_params, needs_layout_passes=False)


def _proj_body(xp_ref, xa_ref, wp_ref, bp_ref, wa_ref, ba_ref, wkr_ref,
               wvr_ref, prw_ref, prc_ref, qp_ref, kv_ref):
    xp = xp_ref[...]
    xa = xa_ref[...]
    wp = wp_ref[...]
    bp = bp_ref[...]
    wa = wa_ref[...]
    ba = ba_ref[...]
    wkr = wkr_ref[...]
    wvr = wvr_ref[...]
    prw = prw_ref[...]
    prc = prc_ref[...]

    scale = 1.0 / math.sqrt(DH)
    qp_ref[...] = (jnp.dot(xp, wp[:, F:2 * F],
                           preferred_element_type=jnp.float32)
                   + bp[:, F:2 * F]) * scale

    def fold(w, b, rel, et, pr):
        cols, bcols = [], []
        for h in range(H):
            r = rel[2 * h + et]
            wh = jnp.dot(w[:, h * DH:(h + 1) * DH], r,
                         preferred_element_type=jnp.float32)
            bh = jnp.dot(b[:, h * DH:(h + 1) * DH], r,
                         preferred_element_type=jnp.float32)
            if pr is not None:
                s = pr[0:1, h:h + 1]
                wh = wh * s
                bh = bh * s
            cols.append(wh)
            bcols.append(bh)
        return jnp.concatenate(cols, axis=1), jnp.concatenate(bcols, axis=1)

    wka, bka = fold(wa[:, :F], ba[:, :F], wkr, 0, prw)
    wkp, bkp = fold(wp[:, :F], bp[:, :F], wkr, 1, prc)
    wva, bva = fold(wa[:, 2 * F:], ba[:, 2 * F:], wvr, 0, None)
    wvp, bvp = fold(wp[:, 2 * F:], bp[:, 2 * F:], wvr, 1, None)

    kv_ref[:N_AUTHOR, :] = jnp.concatenate(
        [jnp.dot(xa, wka, preferred_element_type=jnp.float32) + bka,
         jnp.dot(xa, wva, preferred_element_type=jnp.float32) + bva], axis=1)
    kv_ref[N_AUTHOR:, :] = jnp.concatenate(
        [jnp.dot(xp, wkp, preferred_element_type=jnp.float32) + bkp,
         jnp.dot(xp, wvp, preferred_element_type=jnp.float32) + bvp], axis=1)


_proj = pl.pallas_call(
    _proj_body,
    out_shape=[
        jax.ShapeDtypeStruct((N_PAPER, F), jnp.float32),
        jax.ShapeDtypeStruct((N_TOT, 2 * F), jnp.float32),
    ],
)


GW = 80


def _sc_gather(table, idx2d, width):
    @functools.partial(pl.kernel,
                       out_type=jax.ShapeDtypeStruct((E, width), jnp.float32),
                       mesh=_sc_mesh)
    def body(table_hbm, idx_hbm, out_hbm):
        def step(i_vmem, o_vmem):
            pltpu.sync_copy(table_hbm.at[i_vmem.at[0]], o_vmem)

        pltpu.emit_pipeline(
            step,
            grid=(E // GW,),
            in_specs=[pl.BlockSpec((1, GW), lambda i: (i, 0))],
            out_specs=[pl.BlockSpec((GW, width), lambda i: (i, 0))],
            core_axis_name=("c", "s"),
            dimension_semantics=(pltpu.PARALLEL,),
        )(idx_hbm, out_hbm)

    return body(table, idx2d)


_BC = 4000


def _edge_body(qi_ref, kvj_ref, m_ref, mt_ref, msg_ref, exb_ref):
    p = qi_ref[...] * kvj_ref[:, :F]
    ex = jnp.exp(jnp.dot(p, m_ref[...], preferred_element_type=jnp.float32))
    exb = jnp.dot(ex, mt_ref[...], preferred_element_type=jnp.float32)
    msg_ref[...] = exb * kvj_ref[:, F:]
    exb_ref[...] = exb


_edge = pl.pallas_call(
    _edge_body,
    grid=(E // _BC,),
    in_specs=[
        pl.BlockSpec((_BC, F), lambda i: (i, 0)),
        pl.BlockSpec((_BC, 2 * F), lambda i: (i, 0)),
        pl.BlockSpec((F, H), lambda i: (0, 0)),
        pl.BlockSpec((H, F), lambda i: (0, 0)),
    ],
    out_specs=[
        pl.BlockSpec((_BC, F), lambda i: (i, 0)),
        pl.BlockSpec((_BC, F), lambda i: (i, 0)),
    ],
    out_shape=[
        jax.ShapeDtypeStruct((E, F), jnp.float32),
        jax.ShapeDtypeStruct((E, F), jnp.float32),
    ],
)


SCHUNK = 40
SKBUF = 2
SCHUNKS_PER_W = E // (SCHUNK * NW)


def _sc_scatter(msg, exb, dst, zrows):
    scratch = ([pltpu.VMEM((SCHUNK,), jnp.int32)] * SKBUF
               + [pltpu.VMEM((SCHUNK, F), jnp.float32)] * SKBUF
               + [pltpu.VMEM((SCHUNK, F), jnp.float32)] * SKBUF
               + [pltpu.VMEM_SHARED((ACC_ROWS, F), jnp.float32)]
               + [pltpu.VMEM_SHARED((ACC_ROWS, F), jnp.float32)]
               + [pltpu.SemaphoreType.DMA] * (3 * SKBUF))

    @functools.partial(
        pl.kernel,
        out_type=[
            jax.ShapeDtypeStruct((NC, ACC_ROWS, F), jnp.float32),
            jax.ShapeDtypeStruct((NC, ACC_ROWS, F), jnp.float32),
        ],
        mesh=_sc_mesh,
        scratch_types=scratch)
    def body(msg_hbm, exb_hbm, idx_hbm, z_hbm, out_hbm, outs_hbm, *s):
        idx_b = s[0:SKBUF]
        row_b = s[SKBUF:2 * SKBUF]
        exr_b = s[2 * SKBUF:3 * SKBUF]
        acc = s[3 * SKBUF]
        acc_s = s[3 * SKBUF + 1]
        sem_i = s[3 * SKBUF + 2:4 * SKBUF + 2]
        sem_m = s[4 * SKBUF + 2:5 * SKBUF + 2]
        sem_e = s[5 * SKBUF + 2:6 * SKBUF + 2]
        sid = lax.axis_index("s")
        cid = lax.axis_index("c")
        wid = sid * NC + cid
        base = wid * SCHUNKS_PER_W

        pltpu.sync_copy(z_hbm,
                        acc.at[pl.ds(sid * ROWS_PER_TILE, ROWS_PER_TILE)])
        pltpu.sync_copy(z_hbm,
                        acc_s.at[pl.ds(sid * ROWS_PER_TILE, ROWS_PER_TILE)])
        plsc.subcore_barrier()

        def loads_start(b, g):
            pltpu.async_copy(idx_hbm.at[pl.ds((base + g) * SCHUNK, SCHUNK)],
                             idx_b[b], sem_i[b])
            pltpu.async_copy(msg_hbm.at[pl.ds((base + g) * SCHUNK, SCHUNK)],
                             row_b[b], sem_m[b])
            pltpu.async_copy(exb_hbm.at[pl.ds((base + g) * SCHUNK, SCHUNK)],
                             exr_b[b], sem_e[b])

        def loads_wait(b):
            pltpu.make_async_copy(idx_hbm.at[pl.ds(0, SCHUNK)], idx_b[b],
                                  sem_i[b]).wait()
            pltpu.make_async_copy(msg_hbm.at[pl.ds(0, SCHUNK)], row_b[b],
                                  sem_m[b]).wait()
            pltpu.make_async_copy(exb_hbm.at[pl.ds(0, SCHUNK)], exr_b[b],
                                  sem_e[b]).wait()

        for b in range(SKBUF):
            loads_start(b, b)

        @pl.loop(0, SCHUNKS_PER_W, step=SKBUF)
        def _(g0):
            for b in range(SKBUF):
                loads_wait(b)
                pltpu.sync_copy(row_b[b], acc.at[idx_b[b]], add=True)
                pltpu.sync_copy(exr_b[b], acc_s.at[idx_b[b]], add=True)

                @pl.when(g0 + SKBUF < SCHUNKS_PER_W)
                def _():
                    loads_start(b, g0 + b + SKBUF)

        plsc.subcore_barrier()
        pltpu.sync_copy(acc.at[pl.ds(sid * ROWS_PER_TILE, ROWS_PER_TILE)],
                        out_hbm.at[cid, pl.ds(sid * ROWS_PER_TILE,
                                              ROWS_PER_TILE)])
        pltpu.sync_copy(acc_s.at[pl.ds(sid * ROWS_PER_TILE, ROWS_PER_TILE)],
                        outs_hbm.at[cid, pl.ds(sid * ROWS_PER_TILE,
                                               ROWS_PER_TILE)])

    return body(msg, exb, dst, zrows)


def _final_body(acc_ref, s_ref, xp_ref, wout_ref, bout_ref, skip_ref,
                wcls_ref, bcls_ref, out_ref):
    num = acc_ref[0, :N_PAPER, :] + acc_ref[1, :N_PAPER, :]
    den = s_ref[0, :N_PAPER, :] + s_ref[1, :N_PAPER, :]
    agg = num / (den + 1e-16)
    gel = 0.5 * agg * (1.0 + lax.erf(agg * (1.0 / math.sqrt(2.0))))
    hh = jnp.dot(gel, wout_ref[...],
                 preferred_element_type=jnp.float32) + bout_ref[...]
    al = jax.nn.sigmoid(skip_ref[...])
    o2 = al * hh + (1.0 - al) * xp_ref[...]
    out_ref[...] = jnp.dot(o2, wcls_ref[...],
                           preferred_element_type=jnp.float32) + bcls_ref[...]


_final = pl.pallas_call(
    _final_body,
    out_shape=jax.ShapeDtypeStruct((N_PAPER, NUM_CLASSES), jnp.float32),
)


def kernel(x_paper, x_author, edge_index_writes, edge_index_cites,
           W_kqv_paper, b_kqv_paper, W_kqv_author, b_kqv_author, W_k_rel,
           W_v_rel, W_out_paper, b_out_paper, skip_paper, p_rel_writes,
           p_rel_cites, W_cls, b_cls):
    src = jnp.concatenate([edge_index_writes[0],
                           edge_index_cites[0] + N_AUTHOR])
    dst = jnp.concatenate([edge_index_writes[1], edge_index_cites[1]])

    qp, kv = _proj(x_paper, x_author, W_kqv_paper,
                   b_kqv_paper.reshape(1, -1), W_kqv_author,
                   b_kqv_author.reshape(1, -1), W_k_rel, W_v_rel,
                   p_rel_writes, p_rel_cites)

    dst_g = dst.reshape(E // GW, GW)
    src_g = src.reshape(E // GW, GW)
    qi = _sc_gather(qp, dst_g, F)
    kvj = _sc_gather(kv, src_g, 2 * F)

    m = (jnp.arange(F)[:, None] // DH ==
         jnp.arange(H)[None, :]).astype(jnp.float32)
    msg, exb = _edge(qi, kvj, m, m.T)

    acc, acc_s = _sc_scatter(msg, exb, dst,
                             jnp.zeros((ROWS_PER_TILE, F), jnp.float32))

    return _final(acc, acc_s, x_paper, W_out_paper,
                  b_out_paper.reshape(1, -1),
                  jnp.broadcast_to(skip_paper.reshape(1, 1), (1, F)),
                  W_cls, b_cls.reshape(1, -1))


# speedup vs baseline: 9.6104x; 1.0004x over previous
import functools
import math

import jax
import jax.numpy as jnp
from jax import lax
from jax.experimental import pallas as pl
from jax.experimental.pallas import tpu as pltpu
from jax.experimental.pallas import tpu_sc as plsc

N_PAPER = 6000
N_AUTHOR = 4000
N_TOT = N_PAPER + N_AUTHOR
F = 128
H = 4
DH = 32
E = 320000
NUM_CLASSES = 3

NC = 2
NS = 16
NW = NC * NS
ACC_ROWS = 6016
ROWS_PER_TILE = ACC_ROWS // NS
SCHUNK = 40
SKBUF = 2
SCHUNKS_PER_W = E // (SCHUNK * NW)

_sc_mesh = plsc.VectorSubcoreMesh(core_axis_name="c", subcore_axis_name="s")


def _proj_body(xp_ref, xa_ref, wp_ref, bp_ref, wa_ref, ba_ref, wkr_ref,
               wvr_ref, prw_ref, prc_ref, qp_ref, kv_ref):
    xp = xp_ref[...]
    xa = xa_ref[...]
    wp = wp_ref[...]
    bp = bp_ref[...]
    wa = wa_ref[...]
    ba = ba_ref[...]
    wkr = wkr_ref[...]
    wvr = wvr_ref[...]
    prw = prw_ref[...]
    prc = prc_ref[...]

    scale = 1.0 / math.sqrt(DH)
    qp_ref[...] = (jnp.dot(xp, wp[:, F:2 * F],
                           preferred_element_type=jnp.float32)
                   + bp[:, F:2 * F]) * scale

    def fold(w, b, rel, et, pr):
        cols, bcols = [], []
        for h in range(H):
            r = rel[2 * h + et]
            wh = jnp.dot(w[:, h * DH:(h + 1) * DH], r,
                         preferred_element_type=jnp.float32)
            bh = jnp.dot(b[:, h * DH:(h + 1) * DH], r,
                         preferred_element_type=jnp.float32)
            if pr is not None:
                s = pr[0:1, h:h + 1]
                wh = wh * s
                bh = bh * s
            cols.append(wh)
            bcols.append(bh)
        return jnp.concatenate(cols, axis=1), jnp.concatenate(bcols, axis=1)

    wka, bka = fold(wa[:, :F], ba[:, :F], wkr, 0, prw)
    wkp, bkp = fold(wp[:, :F], bp[:, :F], wkr, 1, prc)
    wva, bva = fold(wa[:, 2 * F:], ba[:, 2 * F:], wvr, 0, None)
    wvp, bvp = fold(wp[:, 2 * F:], bp[:, 2 * F:], wvr, 1, None)

    kv_ref[:N_AUTHOR, :] = jnp.concatenate(
        [jnp.dot(xa, wka, preferred_element_type=jnp.float32) + bka,
         jnp.dot(xa, wva, preferred_element_type=jnp.float32) + bva], axis=1)
    kv_ref[N_AUTHOR:, :] = jnp.concatenate(
        [jnp.dot(xp, wkp, preferred_element_type=jnp.float32) + bkp,
         jnp.dot(xp, wvp, preferred_element_type=jnp.float32) + bvp], axis=1)


_proj = pl.pallas_call(
    _proj_body,
    out_shape=[
        jax.ShapeDtypeStruct((N_PAPER, F), jnp.float32),
        jax.ShapeDtypeStruct((N_TOT, 2 * F), jnp.float32),
    ],
)


GW = 80


def _sc_gather(table, idx2d, width):
    @functools.partial(pl.kernel,
                       out_type=jax.ShapeDtypeStruct((E, width), jnp.float32),
                       mesh=_sc_mesh)
    def body(table_hbm, idx_hbm, out_hbm):
        def step(i_vmem, o_vmem):
            pltpu.sync_copy(table_hbm.at[i_vmem.at[0]], o_vmem)

        pltpu.emit_pipeline(
            step,
            grid=(E // GW,),
            in_specs=[pl.BlockSpec((1, GW), lambda i: (i, 0))],
            out_specs=[pl.BlockSpec((GW, width), lambda i: (i, 0))],
            core_axis_name=("c", "s"),
            dimension_semantics=(pltpu.PARALLEL,),
        )(idx_hbm, out_hbm)

    return body(table, idx2d)


_BC = 4000


def _edge_body(qi_ref, kvj_ref, m_ref, mt_ref, msg_ref, exb_ref):
    p = qi_ref[...] * kvj_ref[:, :F]
    ex = jnp.exp(jnp.dot(p, m_ref[...], preferred_element_type=jnp.float32))
    exb = jnp.dot(ex, mt_ref[...], preferred_element_type=jnp.float32)
    msg_ref[...] = exb * kvj_ref[:, F:]
    exb_ref[...] = exb


_edge = pl.pallas_call(
    _edge_body,
    grid=(E // _BC,),
    in_specs=[
        pl.BlockSpec((_BC, F), lambda i: (i, 0)),
        pl.BlockSpec((_BC, 2 * F), lambda i: (i, 0)),
        pl.BlockSpec((F, H), lambda i: (0, 0)),
        pl.BlockSpec((H, F), lambda i: (0, 0)),
    ],
    out_specs=[
        pl.BlockSpec((_BC, F), lambda i: (i, 0)),
        pl.BlockSpec((_BC, F), lambda i: (i, 0)),
    ],
    out_shape=[
        jax.ShapeDtypeStruct((E, F), jnp.float32),
        jax.ShapeDtypeStruct((E, F), jnp.float32),
    ],
)


def _sc_scatter(msg, exb, dst, zrows):
    scratch = ([pltpu.VMEM((SCHUNK,), jnp.int32)] * SKBUF
               + [pltpu.VMEM((SCHUNK, F), jnp.float32)] * SKBUF
               + [pltpu.VMEM((SCHUNK, F), jnp.float32)] * SKBUF
               + [pltpu.VMEM_SHARED((ACC_ROWS, F), jnp.float32)]
               + [pltpu.VMEM_SHARED((ACC_ROWS, F), jnp.float32)]
               + [pltpu.SemaphoreType.DMA] * (3 * SKBUF))

    @functools.partial(
        pl.kernel,
        out_type=[
            jax.ShapeDtypeStruct((NC, ACC_ROWS, F), jnp.float32),
            jax.ShapeDtypeStruct((NC, ACC_ROWS, F), jnp.float32),
        ],
        mesh=_sc_mesh,
        scratch_types=scratch)
    def body(msg_hbm, exb_hbm, idx_hbm, z_hbm, out_hbm, outs_hbm, *s):
        idx_b = s[0:SKBUF]
        row_b = s[SKBUF:2 * SKBUF]
        exr_b = s[2 * SKBUF:3 * SKBUF]
        acc = s[3 * SKBUF]
        acc_s = s[3 * SKBUF + 1]
        sem_i = s[3 * SKBUF + 2:4 * SKBUF + 2]
        sem_m = s[4 * SKBUF + 2:5 * SKBUF + 2]
        sem_e = s[5 * SKBUF + 2:6 * SKBUF + 2]
        sid = lax.axis_index("s")
        cid = lax.axis_index("c")
        wid = sid * NC + cid
        base = wid * SCHUNKS_PER_W

        pltpu.sync_copy(z_hbm,
                        acc.at[pl.ds(sid * ROWS_PER_TILE, ROWS_PER_TILE)])
        pltpu.sync_copy(z_hbm,
                        acc_s.at[pl.ds(sid * ROWS_PER_TILE, ROWS_PER_TILE)])
        plsc.subcore_barrier()

        def loads_start(b, g):
            pltpu.async_copy(idx_hbm.at[pl.ds((base + g) * SCHUNK, SCHUNK)],
                             idx_b[b], sem_i[b])
            pltpu.async_copy(msg_hbm.at[pl.ds((base + g) * SCHUNK, SCHUNK)],
                             row_b[b], sem_m[b])
            pltpu.async_copy(exb_hbm.at[pl.ds((base + g) * SCHUNK, SCHUNK)],
                             exr_b[b], sem_e[b])

        def loads_wait(b):
            pltpu.make_async_copy(idx_hbm.at[pl.ds(0, SCHUNK)], idx_b[b],
                                  sem_i[b]).wait()
            pltpu.make_async_copy(msg_hbm.at[pl.ds(0, SCHUNK)], row_b[b],
                                  sem_m[b]).wait()
            pltpu.make_async_copy(exb_hbm.at[pl.ds(0, SCHUNK)], exr_b[b],
                                  sem_e[b]).wait()

        for b in range(SKBUF):
            loads_start(b, b)

        @pl.loop(0, SCHUNKS_PER_W, step=SKBUF)
        def _(g0):
            for b in range(SKBUF):
                loads_wait(b)
                pltpu.sync_copy(row_b[b], acc.at[idx_b[b]], add=True)
                pltpu.sync_copy(exr_b[b], acc_s.at[idx_b[b]], add=True)

                @pl.when(g0 + SKBUF < SCHUNKS_PER_W)
                def _():
                    loads_start(b, g0 + b + SKBUF)

        plsc.subcore_barrier()
        pltpu.sync_copy(acc.at[pl.ds(sid * ROWS_PER_TILE, ROWS_PER_TILE)],
                        out_hbm.at[cid, pl.ds(sid * ROWS_PER_TILE,
                                              ROWS_PER_TILE)])
        pltpu.sync_copy(acc_s.at[pl.ds(sid * ROWS_PER_TILE, ROWS_PER_TILE)],
                        outs_hbm.at[cid, pl.ds(sid * ROWS_PER_TILE,
                                               ROWS_PER_TILE)])

    return body(msg, exb, dst, zrows)


def _final_body(acc_ref, s_ref, xp_ref, wout_ref, bout_ref, skip_ref,
                wcls_ref, bcls_ref, out_ref):
    num = acc_ref[0, :N_PAPER, :] + acc_ref[1, :N_PAPER, :]
    den = s_ref[0, :N_PAPER, :] + s_ref[1, :N_PAPER, :]
    agg = num / (den + 1e-16)
    gel = 0.5 * agg * (1.0 + lax.erf(agg * (1.0 / math.sqrt(2.0))))
    hh = jnp.dot(gel, wout_ref[...],
                 preferred_element_type=jnp.float32) + bout_ref[...]
    al = jax.nn.sigmoid(skip_ref[...])
    o2 = al * hh + (1.0 - al) * xp_ref[...]
    out_ref[...] = jnp.dot(o2, wcls_ref[...],
                           preferred_element_type=jnp.float32) + bcls_ref[...]


_final = pl.pallas_call(
    _final_body,
    out_shape=jax.ShapeDtypeStruct((N_PAPER, NUM_CLASSES), jnp.float32),
)


def kernel(x_paper, x_author, edge_index_writes, edge_index_cites,
           W_kqv_paper, b_kqv_paper, W_kqv_author, b_kqv_author, W_k_rel,
           W_v_rel, W_out_paper, b_out_paper, skip_paper, p_rel_writes,
           p_rel_cites, W_cls, b_cls):
    src = jnp.concatenate([edge_index_writes[0],
                           edge_index_cites[0] + N_AUTHOR])
    dst = jnp.concatenate([edge_index_writes[1], edge_index_cites[1]])

    qp, kv = _proj(x_paper, x_author, W_kqv_paper,
                   b_kqv_paper.reshape(1, -1), W_kqv_author,
                   b_kqv_author.reshape(1, -1), W_k_rel, W_v_rel,
                   p_rel_writes, p_rel_cites)

    dst_g = dst.reshape(E // GW, GW)
    src_g = src.reshape(E // GW, GW)
    qi = _sc_gather(qp, dst_g, F)
    kvj = _sc_gather(kv, src_g, 2 * F)

    m = (jnp.arange(F)[:, None] // DH ==
         jnp.arange(H)[None, :]).astype(jnp.float32)
    msg, exb = _edge(qi, kvj, m, m.T)

    acc, acc_s = _sc_scatter(msg, exb, dst,
                             jnp.zeros((ROWS_PER_TILE, F), jnp.float32))

    return _final(acc, acc_s, x_paper, W_out_paper,
                  b_out_paper.reshape(1, -1),
                  jnp.broadcast_to(skip_paper.reshape(1, 1), (1, F)),
                  W_cls, b_cls.reshape(1, -1))
